# Initial kernel scaffold; baseline (speedup 1.0000x reference)
import functools

import jax
import jax.numpy as jnp
from jax import lax
from jax.experimental import pallas as pl
from jax.experimental.pallas import tpu as pltpu
from jax.experimental.pallas import tpu_sc as plsc

N = 10000
E = 320000
C = 128
ROWS = E // C
NW = 32
NS = 16
RPS = N // NS
LANES = 16


def _agg_body(D, h_hbm, src_hbm, dst_hbm, wb_hbm, out_hbm,
              rows_v, src_v, dst_v, wb_v, acc_sh, sem):
    cid = lax.axis_index("c")
    sid = lax.axis_index("s")
    w = sid * 2 + cid

    zero = jnp.zeros((LANES,), jnp.float32)
    G = D // LANES

    def zrow(r, _):
        for g in range(G):
            rows_v[r, pl.ds(g * LANES, LANES)] = zero
        return 0
    lax.fori_loop(0, C, zrow, 0)
    base = sid * RPS
    for k in range(4):
        pltpu.sync_copy(rows_v, acc_sh.at[pl.ds(base + k * C, C)])
    pltpu.sync_copy(rows_v.at[pl.ds(0, RPS - 4 * C)],
                    acc_sh.at[pl.ds(base + 4 * C, RPS - 4 * C)])
    plsc.subcore_barrier()

    nrows = jnp.where(w < ROWS - (ROWS // NW) * NW, ROWS // NW + 1, ROWS // NW)

    def chunk(j, _):
        row = w + j * NW
        pltpu.sync_copy(src_hbm.at[row], src_v.at[0])
        pltpu.sync_copy(dst_hbm.at[row], dst_v.at[0])
        pltpu.sync_copy(wb_hbm.at[row], wb_v)
        pltpu.async_copy(h_hbm.at[src_v.at[0]], rows_v, sem).wait()

        def medge(e, _):
            wvec = wb_v[e, :]
            for g in range(G):
                sl = pl.ds(g * LANES, LANES)
                rows_v[e, sl] = rows_v[e, sl] * wvec
            return 0
        lax.fori_loop(0, C, medge, 0)

        pltpu.sync_copy(rows_v, acc_sh.at[dst_v.at[0]], add=True)
        return 0
    lax.fori_loop(0, nrows, chunk, 0)
    plsc.subcore_barrier()

    for k in range(5):
        cnt = C if k < 4 else RPS - 4 * C
        sl_acc = pl.ds(base + k * C, cnt)
        pltpu.sync_copy(acc_sh.at[sl_acc], rows_v.at[pl.ds(0, cnt)])
        pltpu.sync_copy(rows_v.at[pl.ds(0, cnt)], out_hbm.at[cid, sl_acc])


def _make_agg(D):
    mesh = plsc.VectorSubcoreMesh(core_axis_name="c", subcore_axis_name="s")
    return pl.kernel(
        functools.partial(_agg_body, D),
        out_type=jax.ShapeDtypeStruct((2, N, D), jnp.float32),
        mesh=mesh,
        scratch_types=[
            pltpu.VMEM((C, D), jnp.float32),
            pltpu.VMEM((1, C), jnp.int32),
            pltpu.VMEM((1, C), jnp.int32),
            pltpu.VMEM((C, LANES), jnp.float32),
            pltpu.VMEM_SHARED((N, D), jnp.float32),
            pltpu.SemaphoreType.DMA,
        ],
        name=f"gcn_agg_{D}",
    )


_agg = {128: _make_agg(128), 64: _make_agg(64)}


def _mm_first(x, W, b):
    def body(x_ref, w_ref, b_ref, o_ref):
        o_ref[...] = jnp.dot(x_ref[...], w_ref[...],
                             preferred_element_type=jnp.float32) + b_ref[...]
    return pl.pallas_call(
        body,
        out_shape=jax.ShapeDtypeStruct((N, W.shape[1]), jnp.float32),
        name="gcn_mm0",
    )(x, W, b)


def _mm_mid(p, W, b):
    def body(p_ref, w_ref, b_ref, o_ref):
        h = jnp.maximum(p_ref[0] + p_ref[1], 0.0)
        o_ref[...] = jnp.dot(h, w_ref[...],
                             preferred_element_type=jnp.float32) + b_ref[...]
    return pl.pallas_call(
        body,
        out_shape=jax.ShapeDtypeStruct((N, W.shape[1]), jnp.float32),
        name="gcn_mm",
    )(p, W, b)


def _final_add(p):
    def body(p_ref, o_ref):
        o_ref[...] = p_ref[0] + p_ref[1]
    return pl.pallas_call(
        body,
        out_shape=jax.ShapeDtypeStruct((N, p.shape[2]), jnp.float32),
        name="gcn_final_add",
    )(p)


def kernel(x, edge_index, edge_weight, W0, b0, W1, b1, W2, b2, W3, b3, W4, b4):
    src2 = edge_index[0].reshape(ROWS, C)
    dst2 = edge_index[1].reshape(ROWS, C)
    wb = jnp.broadcast_to(edge_weight[:, None], (E, LANES)).reshape(ROWS, C, LANES)

    h = _mm_first(x, W0, b0.reshape(1, -1))
    p = _agg[128](h, src2, dst2, wb)
    for (W, b) in ((W1, b1), (W2, b2), (W3, b3)):
        h = _mm_mid(p, W, b.reshape(1, -1))
        p = _agg[128](h, src2, dst2, wb)
    h = _mm_mid(p, W4, b4.reshape(1, -1))
    p = _agg[64](h, src2, dst2, wb)
    return _final_add(p)


# speedup vs baseline: 3.6384x; 3.6384x over previous
import functools

import jax
import jax.numpy as jnp
from jax import lax
from jax.experimental import pallas as pl
from jax.experimental.pallas import tpu as pltpu
from jax.experimental.pallas import tpu_sc as plsc

N = 10000
E = 320000
C = 128
ROWS = E // C
NW = 32
NS = 16
QUOTA = 624
LANES = 16


def _agg_body(D, h_hbm, src_hbm, dst_hbm, wb_hbm, out_hbm,
              rows_v, src_v, dst_v, wb_v, acc_sh, sem):
    cid = lax.axis_index("c")
    sid = lax.axis_index("s")
    w = sid * 2 + cid

    zero = jnp.zeros((LANES,), jnp.float32)
    G = D // LANES

    def zrow(r, _):
        for g in range(G):
            rows_v[r, pl.ds(g * LANES, LANES)] = zero
        return 0
    lax.fori_loop(0, C, zrow, 0)
    base = sid * QUOTA
    for k in range(4):
        pltpu.sync_copy(rows_v, acc_sh.at[pl.ds(base + k * C, C)])
    pltpu.sync_copy(rows_v.at[pl.ds(0, QUOTA - 4 * C)],
                    acc_sh.at[pl.ds(base + 4 * C, QUOTA - 4 * C)])

    @pl.when(sid == NS - 1)
    def _():
        pltpu.sync_copy(rows_v.at[pl.ds(0, N - NS * QUOTA)],
                        acc_sh.at[pl.ds(NS * QUOTA, N - NS * QUOTA)])
    plsc.subcore_barrier()

    nrows = jnp.where(w < ROWS - (ROWS // NW) * NW, ROWS // NW + 1, ROWS // NW)

    def chunk(j, _):
        row = w + j * NW
        pltpu.sync_copy(src_hbm.at[row], src_v)
        pltpu.sync_copy(dst_hbm.at[row], dst_v)
        pltpu.sync_copy(wb_hbm.at[row], wb_v)
        pltpu.async_copy(h_hbm.at[src_v.at[0]], rows_v, sem).wait()

        def medge(e, _):
            wvec = wb_v[e, :]
            for g in range(G):
                sl = pl.ds(g * LANES, LANES)
                rows_v[e, sl] = rows_v[e, sl] * wvec
            return 0
        lax.fori_loop(0, C, medge, 0)

        pltpu.sync_copy(rows_v, acc_sh.at[dst_v.at[0]], add=True)
        return 0
    lax.fori_loop(0, nrows, chunk, 0)
    plsc.subcore_barrier()

    for k in range(5):
        cnt = C if k < 4 else QUOTA - 4 * C
        sl_acc = pl.ds(base + k * C, cnt)
        pltpu.sync_copy(acc_sh.at[sl_acc], rows_v.at[pl.ds(0, cnt)])
        pltpu.sync_copy(rows_v.at[pl.ds(0, cnt)], out_hbm.at[cid, sl_acc])

    @pl.when(sid == NS - 1)
    def _():
        tail = pl.ds(NS * QUOTA, N - NS * QUOTA)
        pltpu.sync_copy(acc_sh.at[tail], rows_v.at[pl.ds(0, N - NS * QUOTA)])
        pltpu.sync_copy(rows_v.at[pl.ds(0, N - NS * QUOTA)],
                        out_hbm.at[cid, tail])


def _make_agg(D):
    mesh = plsc.VectorSubcoreMesh(core_axis_name="c", subcore_axis_name="s")
    return pl.kernel(
        functools.partial(_agg_body, D),
        out_type=jax.ShapeDtypeStruct((2, N, D), jnp.float32),
        mesh=mesh,
        scratch_types=[
            pltpu.VMEM((C, D), jnp.float32),
            pltpu.VMEM((1, C), jnp.int32),
            pltpu.VMEM((1, C), jnp.int32),
            pltpu.VMEM((C, LANES), jnp.float32),
            pltpu.VMEM_SHARED((N, D), jnp.float32),
            pltpu.SemaphoreType.DMA,
        ],
        name=f"gcn_agg_{D}",
    )


_agg128 = _make_agg(128)


def _mm_first(x, W, b):
    def body(x_ref, w_ref, b_ref, o_ref):
        o_ref[...] = jnp.dot(x_ref[...], w_ref[...],
                             preferred_element_type=jnp.float32) + b_ref[...]
    return pl.pallas_call(
        body,
        out_shape=jax.ShapeDtypeStruct((N, W.shape[1]), jnp.float32),
        name="gcn_mm0",
    )(x, W, b)


def _mm_mid(p, W, b):
    def body(p_ref, w_ref, b_ref, o_ref):
        h = jnp.maximum(p_ref[0] + p_ref[1], 0.0)
        o_ref[...] = jnp.dot(h, w_ref[...],
                             preferred_element_type=jnp.float32) + b_ref[...]
    return pl.pallas_call(
        body,
        out_shape=jax.ShapeDtypeStruct((N, W.shape[1]), jnp.float32),
        name="gcn_mm",
    )(p, W, b)


def _final_add(p, ncols):
    def body(p_ref, o_ref):
        o_ref[...] = p_ref[0, :, 0:ncols] + p_ref[1, :, 0:ncols]
    return pl.pallas_call(
        body,
        out_shape=jax.ShapeDtypeStruct((N, ncols), jnp.float32),
        name="gcn_final_add",
    )(p)


def kernel(x, edge_index, edge_weight, W0, b0, W1, b1, W2, b2, W3, b3, W4, b4):
    src2 = edge_index[0].reshape(ROWS, 1, C)
    dst2 = edge_index[1].reshape(ROWS, 1, C)
    wb = jnp.broadcast_to(edge_weight[:, None], (E, LANES)).reshape(ROWS, C, LANES)

    h = _mm_first(x, W0, b0.reshape(1, -1))
    p = _agg128(h, src2, dst2, wb)
    for (W, b) in ((W1, b1), (W2, b2), (W3, b3)):
        h = _mm_mid(p, W, b.reshape(1, -1))
        p = _agg128(h, src2, dst2, wb)
    ncls = W4.shape[1]
    W4p = jnp.pad(W4, ((0, 0), (0, 128 - ncls)))
    b4p = jnp.pad(b4, (0, 128 - ncls))
    h = _mm_mid(p, W4p, b4p.reshape(1, -1))
    p = _agg128(h, src2, dst2, wb)
    return _final_add(p, ncls)
